# baseline (speedup 1.0000x reference)
import jax
import jax.numpy as jnp
from jax.experimental import pallas as pl
from jax.experimental.pallas import tpu as pltpu

_NBUF = 12
_DEPTH = 5
_CHUNK_ROWS = 8000
_MOVIE_ROWS = 2000


def _copy_kernel(u_hbm, uo_hbm, buf, in_sem, out_sem):
    users = u_hbm.shape[0]
    n_chunks = users // _CHUNK_ROWS

    def in_copy(c, slot):
        return pltpu.make_async_copy(
            u_hbm.at[pl.ds(c * _CHUNK_ROWS, _CHUNK_ROWS), :],
            buf.at[slot],
            in_sem.at[slot],
        )

    def out_copy(c, slot):
        return pltpu.make_async_copy(
            buf.at[slot],
            uo_hbm.at[pl.ds(c * _CHUNK_ROWS, _CHUNK_ROWS), :],
            out_sem.at[slot],
        )

    for c in range(n_chunks + _DEPTH):
        if c < n_chunks:
            slot = c % _NBUF
            if c >= _NBUF:
                out_copy(c - _NBUF, slot).wait()
            in_copy(c, slot).start()
        if c >= _DEPTH:
            cc = c - _DEPTH
            pslot = cc % _NBUF
            in_copy(cc, pslot).wait()
            out_copy(cc, pslot).start()
    for c in range(max(0, n_chunks - _NBUF), n_chunks):
        out_copy(c, c % _NBUF).wait()


def _mm_kernel(x_ref, w_ref, b_ref, o_ref):
    o_ref[...] = (
        jnp.dot(x_ref[...], w_ref[...], preferred_element_type=jnp.float32)
        + b_ref[...]
    )


def kernel(movie_x, user_emb_weight, W, b):
    m, k = movie_x.shape
    n = W.shape[1]
    users, d = user_emb_weight.shape
    user_out = pl.pallas_call(
        _copy_kernel,
        in_specs=[pl.BlockSpec(memory_space=pltpu.MemorySpace.HBM)],
        out_specs=pl.BlockSpec(memory_space=pltpu.MemorySpace.HBM),
        out_shape=jax.ShapeDtypeStruct((users, d), jnp.float32),
        scratch_shapes=[
            pltpu.VMEM((_NBUF, _CHUNK_ROWS, d), jnp.float32),
            pltpu.SemaphoreType.DMA((_NBUF,)),
            pltpu.SemaphoreType.DMA((_NBUF,)),
        ],
    )(user_emb_weight)
    movie = pl.pallas_call(
        _mm_kernel,
        grid=(m // _MOVIE_ROWS,),
        in_specs=[
            pl.BlockSpec((_MOVIE_ROWS, k), lambda i: (i, 0)),
            pl.BlockSpec((k, n), lambda i: (0, 0)),
            pl.BlockSpec((n,), lambda i: (0,)),
        ],
        out_specs=pl.BlockSpec((_MOVIE_ROWS, n), lambda i: (i, 0)),
        out_shape=jax.ShapeDtypeStruct((m, n), jnp.float32),
    )(movie_x, W, b)
    return (user_out, movie)
